# Initial kernel scaffold; baseline (speedup 1.0000x reference)
import jax
import jax.numpy as jnp
from jax.experimental import pallas as pl


def kernel(user_emb, item_emb, W_l, b_l, W_r, edge_index):
    raise NotImplementedError("write your pallas kernel here")


# speedup vs baseline: 3.9923x; 3.9923x over previous
import functools

import jax
import jax.numpy as jnp
from jax import lax
from jax.experimental import pallas as pl
from jax.experimental.pallas import tpu as pltpu
from jax.experimental.pallas import tpu_sc as plsc

NU = 25000
NI = 25000
NN = NU + NI
NE = 800000
EMB = 64
HALF = EMB // 2

N2 = 51200
E2 = 819200
EPW = E2 // 16
B = 128
NB = EPW // B
RW = N2 // 16
CW = N2 // 2 // 16


def _sc_aggregate(x_lo, x_hi, src, dst, z32, z1):
  mesh = plsc.VectorSubcoreMesh(core_axis_name="c", subcore_axis_name="s")

  @functools.partial(
      pl.kernel,
      mesh=mesh,
      out_type=[
          jax.ShapeDtypeStruct((2, N2, HALF), jnp.float32),
          jax.ShapeDtypeStruct((N2,), jnp.float32),
      ],
      scratch_types=[
          pltpu.VMEM((B,), jnp.int32),
          pltpu.VMEM((B,), jnp.int32),
          pltpu.VMEM((B, HALF), jnp.float32),
          pltpu.VMEM((B,), jnp.float32),
          pltpu.VMEM_SHARED((N2, HALF), jnp.float32),
          pltpu.VMEM_SHARED((N2,), jnp.float32),
          pltpu.SemaphoreType.DMA,
      ],
      compiler_params=pltpu.CompilerParams(use_tc_tiling_on_sc=False),
  )
  def k(xlo_hbm, xhi_hbm, src_hbm, dst_hbm, z32_hbm, z1_hbm,
        agg_out, cnt_out, src_v, dst_v, rows_v, ones_v, acc_sh, cnt_sh, sem):
    c = lax.axis_index("c")
    s = lax.axis_index("s")

    one16 = jnp.ones((16,), jnp.float32)
    for t in range(B // 16):
      ones_v[pl.ds(t * 16, 16)] = one16

    pltpu.sync_copy(z32_hbm.at[pl.ds(s * RW, RW)], acc_sh.at[pl.ds(s * RW, RW)])
    pltpu.sync_copy(z1_hbm.at[pl.ds(s * RW, RW)], cnt_sh.at[pl.ds(s * RW, RW)])
    plsc.subcore_barrier()

    def run(x_hbm):
      def body(j, carry):
        off = s * EPW + j * B
        pltpu.sync_copy(src_hbm.at[pl.ds(off, B)], src_v)
        pltpu.sync_copy(dst_hbm.at[pl.ds(off, B)], dst_v)
        pltpu.async_copy(x_hbm.at[src_v], rows_v, sem).wait()
        pltpu.sync_copy(rows_v, acc_sh.at[dst_v], add=True)
        pltpu.sync_copy(ones_v, cnt_sh.at[dst_v], add=True)
        return carry
      lax.fori_loop(0, NB, body, 0)

    @pl.when(c == 0)
    def _():
      run(xlo_hbm)

    @pl.when(c == 1)
    def _():
      run(xhi_hbm)

    plsc.subcore_barrier()

    @pl.when(c == 0)
    def _():
      pltpu.sync_copy(acc_sh.at[pl.ds(s * RW, RW)],
                      agg_out.at[0].at[pl.ds(s * RW, RW)])
      pltpu.sync_copy(cnt_sh.at[pl.ds(s * CW, CW)],
                      cnt_out.at[pl.ds(s * CW, CW)])

    @pl.when(c == 1)
    def _():
      pltpu.sync_copy(acc_sh.at[pl.ds(s * RW, RW)],
                      agg_out.at[1].at[pl.ds(s * RW, RW)])
      pltpu.sync_copy(cnt_sh.at[pl.ds(N2 // 2 + s * CW, CW)],
                      cnt_out.at[pl.ds(N2 // 2 + s * CW, CW)])

  return k(x_lo, x_hi, src, dst, z32, z1)


def _tc_epilogue_body(x_ref, a0_ref, a1_ref, cnt_ref, wl_ref, bl_ref, wr_ref,
                      out_ref):
  r = jnp.maximum(cnt_ref[...], 1.0)
  a = jnp.concatenate([a0_ref[...], a1_ref[...]], axis=1) / r
  out_ref[...] = (
      jnp.dot(a, wl_ref[...], preferred_element_type=jnp.float32)
      + bl_ref[...]
      + jnp.dot(x_ref[...], wr_ref[...], preferred_element_type=jnp.float32)
  )


def _tc_epilogue(xp, a0, a1, cnt2, W_l, b_l2, W_r):
  BLK = 1024
  grid = (N2 // BLK,)
  return pl.pallas_call(
      _tc_epilogue_body,
      grid=grid,
      in_specs=[
          pl.BlockSpec((BLK, EMB), lambda i: (i, 0)),
          pl.BlockSpec((BLK, HALF), lambda i: (i, 0)),
          pl.BlockSpec((BLK, HALF), lambda i: (i, 0)),
          pl.BlockSpec((BLK, 1), lambda i: (i, 0)),
          pl.BlockSpec((EMB, EMB), lambda i: (0, 0)),
          pl.BlockSpec((1, EMB), lambda i: (0, 0)),
          pl.BlockSpec((EMB, EMB), lambda i: (0, 0)),
      ],
      out_specs=pl.BlockSpec((BLK, EMB), lambda i: (i, 0)),
      out_shape=jax.ShapeDtypeStruct((N2, EMB), jnp.float32),
  )(xp, a0, a1, cnt2, W_l, b_l2, W_r)


@jax.jit
def kernel(user_emb, item_emb, W_l, b_l, W_r, edge_index):
  x = jnp.concatenate([user_emb, item_emb], axis=0)
  xp = jnp.pad(x, ((0, N2 - NN), (0, 0)))
  x_lo = xp[:, :HALF]
  x_hi = xp[:, HALF:]

  src = jnp.pad(edge_index[0], (0, E2 - NE))
  dst = jnp.pad(edge_index[1], (0, E2 - NE),
                constant_values=N2 - 1)

  z32 = jnp.zeros((N2, HALF), jnp.float32)
  z1 = jnp.zeros((N2,), jnp.float32)

  agg2, cnt = _sc_aggregate(x_lo, x_hi, src, dst, z32, z1)

  out = _tc_epilogue(xp, agg2[0], agg2[1], cnt[:, None], W_l, b_l[None, :],
                     W_r)
  return (out[:NU], out[NU:NN])


# speedup vs baseline: 6.9775x; 1.7477x over previous
import functools

import jax
import jax.numpy as jnp
from jax import lax
from jax.experimental import pallas as pl
from jax.experimental.pallas import tpu as pltpu
from jax.experimental.pallas import tpu_sc as plsc

NU = 25000
NI = 25000
NN = NU + NI
NE = 800000
EMB = 64
HALF = EMB // 2

N2 = 51200
E2 = 819200
EPW = E2 // 16
B = 128
NB = EPW // B
CB = 20
NCH = NB // CB
RW = N2 // 16
CW = N2 // 2 // 16


def _sc_aggregate(x_lo, x_hi, src, dst, z32, z1):
  mesh = plsc.VectorSubcoreMesh(core_axis_name="c", subcore_axis_name="s")

  @functools.partial(
      pl.kernel,
      mesh=mesh,
      out_type=[
          jax.ShapeDtypeStruct((2, N2, HALF), jnp.float32),
          jax.ShapeDtypeStruct((N2,), jnp.float32),
      ],
      scratch_types=[
          pltpu.VMEM((CB * B,), jnp.int32),
          pltpu.VMEM((CB * B,), jnp.int32),
          pltpu.VMEM((CB, B), jnp.int32),
          pltpu.VMEM((CB, B), jnp.int32),
          pltpu.VMEM((B, HALF), jnp.float32),
          pltpu.VMEM((B, HALF), jnp.float32),
          pltpu.VMEM((B,), jnp.float32),
          pltpu.VMEM_SHARED((N2, HALF), jnp.float32),
          pltpu.VMEM_SHARED((N2,), jnp.float32),
          pltpu.SemaphoreType.DMA,
          pltpu.SemaphoreType.DMA,
          pltpu.SemaphoreType.DMA,
          pltpu.SemaphoreType.DMA,
          pltpu.SemaphoreType.DMA,
          pltpu.SemaphoreType.DMA,
          pltpu.SemaphoreType.DMA,
          pltpu.SemaphoreType.DMA,
      ],
      compiler_params=pltpu.CompilerParams(use_tc_tiling_on_sc=False),
  )
  def k(xlo_hbm, xhi_hbm, src_hbm, dst2_hbm, z32_hbm, z1_hbm,
        agg_out, cnt_out, srcb0, srcb1, dstb0, dstb1, rows0, rows1, ones_v,
        acc_sh, cnt_sh, gsem0, gsem1, ssem0, ssem1, csem0, csem1,
        isem0, isem1):
    c = lax.axis_index("c")
    s = lax.axis_index("s")
    srcb = (srcb0, srcb1)
    dstb = (dstb0, dstb1)
    isem = (isem0, isem1)

    one16 = jnp.ones((16,), jnp.float32)
    for t in range(B // 16):
      ones_v[pl.ds(t * 16, 16)] = one16

    pltpu.sync_copy(z32_hbm.at[pl.ds(s * RW, RW)], acc_sh.at[pl.ds(s * RW, RW)])
    pltpu.sync_copy(z1_hbm.at[pl.ds(s * RW, RW)], cnt_sh.at[pl.ds(s * RW, RW)])
    plsc.subcore_barrier()

    def run(x_hbm):
      def ifire(kc, p):
        pltpu.async_copy(
            src_hbm.at[pl.ds(s * EPW + kc * CB * B, CB * B)], srcb[p],
            isem[p])
        pltpu.async_copy(
            dst2_hbm.at[pl.ds(s * NB + kc * CB, CB)], dstb[p], isem[p])

      def iwait(p):
        pltpu.make_async_copy(
            src_hbm.at[pl.ds(0, CB * B)], srcb[p], isem[p]).wait()
        pltpu.make_async_copy(
            dst2_hbm.at[pl.ds(0, CB)], dstb[p], isem[p]).wait()

      def gfire(src_c, la, buf, gsem):
        pltpu.async_copy(x_hbm.at[src_c.at[pl.ds(la * B, B)]], buf, gsem)

      def process_chunk(p):
        src_c, dst_c = srcb[p], dstb[p]

        def step(la, buf, gsem, ssem, csem):
          pltpu.make_async_copy(
              x_hbm.at[src_c.at[pl.ds(0, B)]], buf, gsem).wait()
          pltpu.async_copy(buf, acc_sh.at[dst_c.at[la]], ssem, add=True)
          pltpu.async_copy(ones_v, cnt_sh.at[dst_c.at[la]], csem, add=True)
          pltpu.make_async_copy(buf, acc_sh.at[dst_c.at[la]], ssem).wait()
          pltpu.make_async_copy(ones_v, cnt_sh.at[dst_c.at[la]], csem).wait()
          @pl.when(la + 2 < CB)
          def _():
            gfire(src_c, la + 2, buf, gsem)

        gfire(src_c, 0, rows0, gsem0)
        gfire(src_c, 1, rows1, gsem1)

        def body(j2, carry):
          step(2 * j2, rows0, gsem0, ssem0, csem0)
          step(2 * j2 + 1, rows1, gsem1, ssem1, csem1)
          return carry
        lax.fori_loop(0, CB // 2, body, 0)

      ifire(0, 0)
      iwait(0)
      for kc in range(NCH):
        p = kc % 2
        if kc + 1 < NCH:
          ifire(kc + 1, 1 - p)
        if kc > 0:
          iwait(p)
        process_chunk(p)

    @pl.when(c == 0)
    def _():
      run(xlo_hbm)

    @pl.when(c == 1)
    def _():
      run(xhi_hbm)

    plsc.subcore_barrier()

    @pl.when(c == 0)
    def _():
      pltpu.sync_copy(acc_sh.at[pl.ds(s * RW, RW)],
                      agg_out.at[0].at[pl.ds(s * RW, RW)])
      pltpu.sync_copy(cnt_sh.at[pl.ds(s * CW, CW)],
                      cnt_out.at[pl.ds(s * CW, CW)])

    @pl.when(c == 1)
    def _():
      pltpu.sync_copy(acc_sh.at[pl.ds(s * RW, RW)],
                      agg_out.at[1].at[pl.ds(s * RW, RW)])
      pltpu.sync_copy(cnt_sh.at[pl.ds(N2 // 2 + s * CW, CW)],
                      cnt_out.at[pl.ds(N2 // 2 + s * CW, CW)])

  return k(x_lo, x_hi, src, dst, z32, z1)


def _tc_epilogue_body(x_ref, a0_ref, a1_ref, cnt_ref, wl_ref, bl_ref, wr_ref,
                      out_ref):
  r = jnp.maximum(cnt_ref[...], 1.0)
  a = jnp.concatenate([a0_ref[...], a1_ref[...]], axis=1) / r
  out_ref[...] = (
      jnp.dot(a, wl_ref[...], preferred_element_type=jnp.float32)
      + bl_ref[...]
      + jnp.dot(x_ref[...], wr_ref[...], preferred_element_type=jnp.float32)
  )


def _tc_epilogue(xp, a0, a1, cnt2, W_l, b_l2, W_r):
  BLK = 1024
  grid = (N2 // BLK,)
  return pl.pallas_call(
      _tc_epilogue_body,
      grid=grid,
      in_specs=[
          pl.BlockSpec((BLK, EMB), lambda i: (i, 0)),
          pl.BlockSpec((BLK, HALF), lambda i: (i, 0)),
          pl.BlockSpec((BLK, HALF), lambda i: (i, 0)),
          pl.BlockSpec((BLK, 1), lambda i: (i, 0)),
          pl.BlockSpec((EMB, EMB), lambda i: (0, 0)),
          pl.BlockSpec((1, EMB), lambda i: (0, 0)),
          pl.BlockSpec((EMB, EMB), lambda i: (0, 0)),
      ],
      out_specs=pl.BlockSpec((BLK, EMB), lambda i: (i, 0)),
      out_shape=jax.ShapeDtypeStruct((N2, EMB), jnp.float32),
  )(xp, a0, a1, cnt2, W_l, b_l2, W_r)


@jax.jit
def kernel(user_emb, item_emb, W_l, b_l, W_r, edge_index):
  x = jnp.concatenate([user_emb, item_emb], axis=0)
  xp = jnp.pad(x, ((0, N2 - NN), (0, 0)))
  x_lo = xp[:, :HALF]
  x_hi = xp[:, HALF:]

  src = jnp.pad(edge_index[0], (0, E2 - NE))
  dst = jnp.pad(edge_index[1], (0, E2 - NE),
                constant_values=N2 - 1)
  dst = dst.reshape(E2 // B, B)

  z32 = jnp.zeros((N2, HALF), jnp.float32)
  z1 = jnp.zeros((N2,), jnp.float32)

  agg2, cnt = _sc_aggregate(x_lo, x_hi, src, dst, z32, z1)

  out = _tc_epilogue(xp, agg2[0], agg2[1], cnt[:, None], W_l, b_l[None, :],
                     W_r)
  return (out[:NU], out[NU:NN])


# speedup vs baseline: 7.6542x; 1.0970x over previous
import functools

import jax
import jax.numpy as jnp
from jax import lax
from jax.experimental import pallas as pl
from jax.experimental.pallas import tpu as pltpu
from jax.experimental.pallas import tpu_sc as plsc

NU = 25000
NI = 25000
NN = NU + NI
NE = 800000
EMB = 64
HALF = EMB // 2

N2 = 51200
E2 = 819200
EPW = E2 // 16
B = 128
NB = EPW // B
CB = 8
NCH = NB // CB
NBUF = 4
RW = N2 // 16
CW = N2 // 2 // 16


def _sc_aggregate(x_lo, x_hi, src, dst2, z32, z1):
  mesh = plsc.VectorSubcoreMesh(core_axis_name="c", subcore_axis_name="s")

  @functools.partial(
      pl.kernel,
      mesh=mesh,
      out_type=[
          jax.ShapeDtypeStruct((2, N2, HALF), jnp.float32),
          jax.ShapeDtypeStruct((N2,), jnp.float32),
      ],
      scratch_types=[
          pltpu.VMEM((CB * B,), jnp.int32),
          pltpu.VMEM((CB * B,), jnp.int32),
          pltpu.VMEM((CB, B), jnp.int32),
          pltpu.VMEM((CB, B), jnp.int32),
          pltpu.VMEM((B, HALF), jnp.float32),
          pltpu.VMEM((B, HALF), jnp.float32),
          pltpu.VMEM((B, HALF), jnp.float32),
          pltpu.VMEM((B, HALF), jnp.float32),
          pltpu.VMEM((B,), jnp.float32),
          pltpu.VMEM_SHARED((N2, HALF), jnp.float32),
          pltpu.VMEM_SHARED((N2,), jnp.float32),
          pltpu.SemaphoreType.DMA,
          pltpu.SemaphoreType.DMA,
          pltpu.SemaphoreType.DMA,
          pltpu.SemaphoreType.DMA,
          pltpu.SemaphoreType.DMA,
          pltpu.SemaphoreType.DMA,
          pltpu.SemaphoreType.DMA,
          pltpu.SemaphoreType.DMA,
          pltpu.SemaphoreType.DMA,
          pltpu.SemaphoreType.DMA,
          pltpu.SemaphoreType.DMA,
          pltpu.SemaphoreType.DMA,
          pltpu.SemaphoreType.DMA,
          pltpu.SemaphoreType.DMA,
      ],
      compiler_params=pltpu.CompilerParams(use_tc_tiling_on_sc=False),
  )
  def k(xlo_hbm, xhi_hbm, src_hbm, dst2_hbm, z32_hbm, z1_hbm,
        agg_out, cnt_out, srcb0, srcb1, dstb0, dstb1, r0, r1, r2, r3,
        ones_v, acc_sh, cnt_sh,
        g0, g1, g2, g3, s0, s1, s2, s3, c0, c1, c2, c3, i0, i1):
    c = lax.axis_index("c")
    s = lax.axis_index("s")
    srcb = (srcb0, srcb1)
    dstb = (dstb0, dstb1)
    rows = (r0, r1, r2, r3)
    gsem = (g0, g1, g2, g3)
    ssem = (s0, s1, s2, s3)
    csem = (c0, c1, c2, c3)
    isem = (i0, i1)

    one16 = jnp.ones((16,), jnp.float32)
    for t in range(B // 16):
      ones_v[pl.ds(t * 16, 16)] = one16

    pltpu.sync_copy(z32_hbm.at[pl.ds(s * RW, RW)], acc_sh.at[pl.ds(s * RW, RW)])
    pltpu.sync_copy(z1_hbm.at[pl.ds(s * RW, RW)], cnt_sh.at[pl.ds(s * RW, RW)])
    plsc.subcore_barrier()

    def ifire(kc, p):
      pltpu.async_copy(
          src_hbm.at[pl.ds(s * EPW + kc * CB * B, CB * B)], srcb[p], isem[p])
      pltpu.async_copy(
          dst2_hbm.at[pl.ds(s * NB + kc * CB, CB)], dstb[p], isem[p])

    def iwait(p):
      pltpu.make_async_copy(
          src_hbm.at[pl.ds(0, CB * B)], srcb[p], isem[p]).wait()
      pltpu.make_async_copy(
          dst2_hbm.at[pl.ds(0, CB)], dstb[p], isem[p]).wait()

    def run(x_hbm):
      def gfire(src_c, la, b):
        pltpu.async_copy(
            x_hbm.at[src_c.at[pl.ds(la * B, B)]], rows[b], gsem[b])

      def gwait(b):
        pltpu.make_async_copy(
            x_hbm.at[srcb0.at[pl.ds(0, B)]], rows[b], gsem[b]).wait()

      def swait(b):
        pltpu.make_async_copy(
            rows[b], acc_sh.at[dstb0.at[0]], ssem[b]).wait()
        pltpu.make_async_copy(
            ones_v, cnt_sh.at[dstb0.at[0]], csem[b]).wait()

      def chunk(kc, p):
        src_c, dst_c = srcb[p], dstb[p]
        src_n = srcb[1 - p]
        for la in range(CB):
          a = kc * CB + la
          b = la % NBUF
          b2 = (la + 2) % NBUF
          gwait(b)
          pltpu.async_copy(rows[b], acc_sh.at[dst_c.at[la]], ssem[b],
                           add=True)
          pltpu.async_copy(ones_v, cnt_sh.at[dst_c.at[la]], csem[b],
                           add=True)

          @pl.when(a >= 2)
          def _():
            swait(b2)

          if la + 2 < CB:
            @pl.when(a + 2 < NB)
            def _():
              gfire(src_c, la + 2, b2)
          else:
            @pl.when(a + 2 < NB)
            def _():
              gfire(src_n, la + 2 - CB, b2)

          if la == 1:
            @pl.when((kc >= 1) & (kc + 1 < NCH))
            def _():
              ifire(kc + 1, 1 - p)
          if la == CB - 3:
            @pl.when(kc + 1 < NCH)
            def _():
              iwait(1 - p)

      ifire(0, 0)
      iwait(0)
      ifire(1, 1)
      gfire(srcb[0], 0, 0)
      gfire(srcb[0], 1, 1)

      def body(kp, carry):
        chunk(2 * kp, 0)
        chunk(2 * kp + 1, 1)
        return carry
      lax.fori_loop(0, NCH // 2, body, 0)

      swait((NB - 2) % NBUF)
      swait((NB - 1) % NBUF)

    @pl.when(c == 0)
    def _():
      run(xlo_hbm)

    @pl.when(c == 1)
    def _():
      run(xhi_hbm)

    plsc.subcore_barrier()

    @pl.when(c == 0)
    def _():
      pltpu.sync_copy(acc_sh.at[pl.ds(s * RW, RW)],
                      agg_out.at[0].at[pl.ds(s * RW, RW)])
      pltpu.sync_copy(cnt_sh.at[pl.ds(s * CW, CW)],
                      cnt_out.at[pl.ds(s * CW, CW)])

    @pl.when(c == 1)
    def _():
      pltpu.sync_copy(acc_sh.at[pl.ds(s * RW, RW)],
                      agg_out.at[1].at[pl.ds(s * RW, RW)])
      pltpu.sync_copy(cnt_sh.at[pl.ds(N2 // 2 + s * CW, CW)],
                      cnt_out.at[pl.ds(N2 // 2 + s * CW, CW)])

  return k(x_lo, x_hi, src, dst2, z32, z1)


def _tc_epilogue_body(x_ref, a0_ref, a1_ref, cnt_ref, wl_ref, bl_ref, wr_ref,
                      out_ref):
  r = jnp.maximum(cnt_ref[...], 1.0)
  a = jnp.concatenate([a0_ref[0], a1_ref[0]], axis=1) / r
  out_ref[...] = (
      jnp.dot(a, wl_ref[...], preferred_element_type=jnp.float32)
      + bl_ref[...]
      + jnp.dot(x_ref[...], wr_ref[...], preferred_element_type=jnp.float32)
  )


def _tc_epilogue(x_half, agg2, cnt2, W_l, b_l2, W_r, row0):
  BLK = 1000
  nblk = 25000 // BLK
  r0 = row0 // BLK
  return pl.pallas_call(
      _tc_epilogue_body,
      grid=(nblk,),
      in_specs=[
          pl.BlockSpec((BLK, EMB), lambda i: (i, 0)),
          pl.BlockSpec((1, BLK, HALF), lambda i, r0=r0: (0, r0 + i, 0)),
          pl.BlockSpec((1, BLK, HALF), lambda i, r0=r0: (1, r0 + i, 0)),
          pl.BlockSpec((BLK, 1), lambda i, r0=r0: (r0 + i, 0)),
          pl.BlockSpec((EMB, EMB), lambda i: (0, 0)),
          pl.BlockSpec((1, EMB), lambda i: (0, 0)),
          pl.BlockSpec((EMB, EMB), lambda i: (0, 0)),
      ],
      out_specs=pl.BlockSpec((BLK, EMB), lambda i: (i, 0)),
      out_shape=jax.ShapeDtypeStruct((25000, EMB), jnp.float32),
  )(x_half, agg2, agg2, cnt2, W_l, b_l2, W_r)


@jax.jit
def kernel(user_emb, item_emb, W_l, b_l, W_r, edge_index):
  x_lo = jnp.concatenate([user_emb[:, :HALF], item_emb[:, :HALF]], axis=0)
  x_hi = jnp.concatenate([user_emb[:, HALF:], item_emb[:, HALF:]], axis=0)

  src = jnp.pad(edge_index[0], (0, E2 - NE))
  dst = jnp.pad(edge_index[1], (0, E2 - NE),
                constant_values=N2 - 1)
  dst2 = dst.reshape(E2 // B, B)

  z32 = jnp.zeros((N2, HALF), jnp.float32)
  z1 = jnp.zeros((N2,), jnp.float32)

  agg2, cnt = _sc_aggregate(x_lo, x_hi, src, dst2, z32, z1)
  cnt2 = cnt[:, None]
  b_l2 = b_l[None, :]

  out_u = _tc_epilogue(user_emb, agg2, cnt2, W_l, b_l2, W_r, 0)
  out_i = _tc_epilogue(item_emb, agg2, cnt2, W_l, b_l2, W_r, NU)
  return (out_u, out_i)


# speedup vs baseline: 8.7355x; 1.1413x over previous
import functools

import jax
import jax.numpy as jnp
from jax import lax
from jax.experimental import pallas as pl
from jax.experimental.pallas import tpu as pltpu
from jax.experimental.pallas import tpu_sc as plsc

NU = 25000
NI = 25000
NN = NU + NI
NE = 800000
EMB = 64
HALF = EMB // 2
Q = 16

N2 = 51200
E2 = 819200
EPW = E2 // 16
B = 128
NB = EPW // B
CB = 8
NCH = NB // CB
NBUF = 4
RW = N2 // 16
CW = N2 // 2 // 16
TR = NN // 16


def _sc_aggregate(x4, src, dst2, z16, z1):
  mesh = plsc.VectorSubcoreMesh(core_axis_name="c", subcore_axis_name="s")

  @functools.partial(
      pl.kernel,
      mesh=mesh,
      out_type=[
          jax.ShapeDtypeStruct((4, N2, Q), jnp.float32),
          jax.ShapeDtypeStruct((N2,), jnp.float32),
      ],
      scratch_types=[
          pltpu.VMEM((CB * B,), jnp.int32),
          pltpu.VMEM((CB * B,), jnp.int32),
          pltpu.VMEM((CB, B), jnp.int32),
          pltpu.VMEM((CB, B), jnp.int32),
          pltpu.VMEM((B, Q), jnp.float32),
          pltpu.VMEM((B, Q), jnp.float32),
          pltpu.VMEM((B, Q), jnp.float32),
          pltpu.VMEM((B, Q), jnp.float32),
          pltpu.VMEM((B,), jnp.float32),
          pltpu.VMEM_SHARED((N2, Q), jnp.float32),
          pltpu.VMEM_SHARED((N2, Q), jnp.float32),
          pltpu.VMEM_SHARED((N2,), jnp.float32),
          pltpu.SemaphoreType.DMA,
          pltpu.SemaphoreType.DMA,
          pltpu.SemaphoreType.DMA,
          pltpu.SemaphoreType.DMA,
          pltpu.SemaphoreType.DMA,
          pltpu.SemaphoreType.DMA,
          pltpu.SemaphoreType.DMA,
          pltpu.SemaphoreType.DMA,
          pltpu.SemaphoreType.DMA,
          pltpu.SemaphoreType.DMA,
          pltpu.SemaphoreType.DMA,
          pltpu.SemaphoreType.DMA,
          pltpu.SemaphoreType.DMA,
          pltpu.SemaphoreType.DMA,
      ],
      compiler_params=pltpu.CompilerParams(use_tc_tiling_on_sc=False),
  )
  def k(x4_hbm, src_hbm, dst2_hbm, z16_hbm, z1_hbm,
        agg_out, cnt_out, srcb0, srcb1, dstb0, dstb1, r0, r1, r2, r3,
        ones_v, tab_sh, acc_sh, cnt_sh,
        g0, g1, g2, g3, s0, s1, s2, s3, c0, c1, c2, c3, i0, i1):
    c = lax.axis_index("c")
    s = lax.axis_index("s")
    srcb = (srcb0, srcb1)
    dstb = (dstb0, dstb1)
    rows = (r0, r1, r2, r3)
    gsem = (g0, g1, g2, g3)
    ssem = (s0, s1, s2, s3)
    csem = (c0, c1, c2, c3)
    isem = (i0, i1)

    one16 = jnp.ones((16,), jnp.float32)
    for t in range(B // 16):
      ones_v[pl.ds(t * 16, 16)] = one16

    def ifire(kc, p):
      pltpu.async_copy(
          src_hbm.at[pl.ds(s * EPW + kc * CB * B, CB * B)], srcb[p], isem[p])
      pltpu.async_copy(
          dst2_hbm.at[pl.ds(s * NB + kc * CB, CB)], dstb[p], isem[p])

    def iwait(p):
      pltpu.make_async_copy(
          src_hbm.at[pl.ds(0, CB * B)], srcb[p], isem[p]).wait()
      pltpu.make_async_copy(
          dst2_hbm.at[pl.ds(0, CB)], dstb[p], isem[p]).wait()

    def run(qq, with_cnt):
      @pl.when(c == 0)
      def _():
        pltpu.sync_copy(x4_hbm.at[pl.ds(s * TR, TR), qq],
                        tab_sh.at[pl.ds(s * TR, TR)])

      @pl.when(c == 1)
      def _():
        pltpu.sync_copy(x4_hbm.at[pl.ds(s * TR, TR), 2 + qq],
                        tab_sh.at[pl.ds(s * TR, TR)])

      pltpu.sync_copy(z16_hbm.at[pl.ds(s * RW, RW)],
                      acc_sh.at[pl.ds(s * RW, RW)])
      if with_cnt:
        pltpu.sync_copy(z1_hbm.at[pl.ds(s * RW, RW)],
                        cnt_sh.at[pl.ds(s * RW, RW)])
      plsc.subcore_barrier()

      def gfire(src_c, la, b):
        pltpu.async_copy(
            tab_sh.at[src_c.at[pl.ds(la * B, B)]], rows[b], gsem[b])

      def gwait(b):
        pltpu.make_async_copy(
            tab_sh.at[srcb0.at[pl.ds(0, B)]], rows[b], gsem[b]).wait()

      def swait(b):
        pltpu.make_async_copy(
            rows[b], acc_sh.at[dstb0.at[0]], ssem[b]).wait()
        if with_cnt:
          pltpu.make_async_copy(
              ones_v, cnt_sh.at[dstb0.at[0]], csem[b]).wait()

      def chunk(kc, p):
        src_c, dst_c = srcb[p], dstb[p]
        src_n = srcb[1 - p]
        for la in range(CB):
          a = kc * CB + la
          b = la % NBUF
          b2 = (la + 2) % NBUF
          gwait(b)
          pltpu.async_copy(rows[b], acc_sh.at[dst_c.at[la]], ssem[b],
                           add=True)
          if with_cnt:
            pltpu.async_copy(ones_v, cnt_sh.at[dst_c.at[la]], csem[b],
                             add=True)

          @pl.when(a >= 2)
          def _():
            swait(b2)

          if la + 2 < CB:
            @pl.when(a + 2 < NB)
            def _():
              gfire(src_c, la + 2, b2)
          else:
            @pl.when(a + 2 < NB)
            def _():
              gfire(src_n, la + 2 - CB, b2)

          if la == 1:
            @pl.when((kc >= 1) & (kc + 1 < NCH))
            def _():
              ifire(kc + 1, 1 - p)
          if la == CB - 3:
            @pl.when(kc + 1 < NCH)
            def _():
              iwait(1 - p)

      ifire(0, 0)
      iwait(0)
      ifire(1, 1)
      gfire(srcb[0], 0, 0)
      gfire(srcb[0], 1, 1)

      def body(kp, carry):
        chunk(2 * kp, 0)
        chunk(2 * kp + 1, 1)
        return carry
      lax.fori_loop(0, NCH // 2, body, 0)

      swait((NB - 2) % NBUF)
      swait((NB - 1) % NBUF)
      plsc.subcore_barrier()

      @pl.when(c == 0)
      def _():
        pltpu.sync_copy(acc_sh.at[pl.ds(s * RW, RW)],
                        agg_out.at[qq].at[pl.ds(s * RW, RW)])

      @pl.when(c == 1)
      def _():
        pltpu.sync_copy(acc_sh.at[pl.ds(s * RW, RW)],
                        agg_out.at[2 + qq].at[pl.ds(s * RW, RW)])

      if with_cnt:
        @pl.when(c == 0)
        def _():
          pltpu.sync_copy(cnt_sh.at[pl.ds(s * CW, CW)],
                          cnt_out.at[pl.ds(s * CW, CW)])

        @pl.when(c == 1)
        def _():
          pltpu.sync_copy(cnt_sh.at[pl.ds(N2 // 2 + s * CW, CW)],
                          cnt_out.at[pl.ds(N2 // 2 + s * CW, CW)])
      plsc.subcore_barrier()

    run(0, True)
    run(1, False)

  return k(x4, src, dst2, z16, z1)


def _tc_epilogue_body(x_ref, a0_ref, a1_ref, a2_ref, a3_ref, cnt_ref,
                      wl_ref, bl_ref, wr_ref, out_ref):
  r = jnp.maximum(cnt_ref[...], 1.0)
  a = jnp.concatenate(
      [a0_ref[0], a1_ref[0], a2_ref[0], a3_ref[0]], axis=1) / r
  out_ref[...] = (
      jnp.dot(a, wl_ref[...], preferred_element_type=jnp.float32)
      + bl_ref[...]
      + jnp.dot(x_ref[...], wr_ref[...], preferred_element_type=jnp.float32)
  )


def _tc_epilogue(x_half, agg4, cnt2, W_l, b_l2, W_r, row0):
  BLK = 1000
  nblk = 25000 // BLK
  r0 = row0 // BLK

  def aspec(kq):
    return pl.BlockSpec((1, BLK, Q),
                        lambda i, kq=kq, r0=r0: (kq, r0 + i, 0))

  return pl.pallas_call(
      _tc_epilogue_body,
      grid=(nblk,),
      in_specs=[
          pl.BlockSpec((BLK, EMB), lambda i: (i, 0)),
          aspec(0), aspec(1), aspec(2), aspec(3),
          pl.BlockSpec((BLK, 1), lambda i, r0=r0: (r0 + i, 0)),
          pl.BlockSpec((EMB, EMB), lambda i: (0, 0)),
          pl.BlockSpec((1, EMB), lambda i: (0, 0)),
          pl.BlockSpec((EMB, EMB), lambda i: (0, 0)),
      ],
      out_specs=pl.BlockSpec((BLK, EMB), lambda i: (i, 0)),
      out_shape=jax.ShapeDtypeStruct((25000, EMB), jnp.float32),
  )(x_half, agg4, agg4, agg4, agg4, cnt2, W_l, b_l2, W_r)


@jax.jit
def kernel(user_emb, item_emb, W_l, b_l, W_r, edge_index):
  x4 = jnp.concatenate([user_emb, item_emb], axis=0).reshape(NN, 4, Q)

  src = jnp.pad(edge_index[0], (0, E2 - NE))
  dst = jnp.pad(edge_index[1], (0, E2 - NE),
                constant_values=N2 - 1)
  dst2 = dst.reshape(E2 // B, B)

  z16 = jnp.zeros((N2, Q), jnp.float32)
  z1 = jnp.zeros((N2,), jnp.float32)

  agg4, cnt = _sc_aggregate(x4, src, dst2, z16, z1)
  cnt2 = cnt[:, None]
  b_l2 = b_l[None, :]

  out_u = _tc_epilogue(user_emb, agg4, cnt2, W_l, b_l2, W_r, 0)
  out_i = _tc_epilogue(item_emb, agg4, cnt2, W_l, b_l2, W_r, NU)
  return (out_u, out_i)
